# baseline (speedup 1.0000x reference)
import jax
import jax.numpy as jnp
from jax.experimental import pallas as pl
from jax.experimental.pallas import tpu as pltpu


def _body(x_ref, cg_ref, wmu_ref, wsig_ref, epsw_ref,
          bmu_ref, bsig_ref, epsb_ref, o_ref):
    w = cg_ref[...] * (wmu_ref[...] + wsig_ref[...] * epsw_ref[...])
    bias = bmu_ref[...] + bsig_ref[...] * epsb_ref[...]
    o_ref[...] = jnp.dot(x_ref[...], w,
                         preferred_element_type=jnp.float32) + bias


def kernel(x, cgamma_t, weight_mu_t, weight_sigma_t, eps_w_t,
           bias_mu_row, bias_sigma_row, eps_b):
    B, I = x.shape
    O = weight_mu_t.shape[1]
    TN = 256
    assert O % TN == 0
    grid = (O // TN,)

    w_spec = pl.BlockSpec((I, TN), lambda n: (0, n))
    row_spec = pl.BlockSpec((1, TN), lambda n: (0, n))

    return pl.pallas_call(
        _body,
        out_shape=jax.ShapeDtypeStruct((B, O), jnp.float32),
        grid=grid,
        in_specs=[pl.BlockSpec((B, I), lambda n: (0, 0)),
                  w_spec, w_spec, w_spec, w_spec,
                  row_spec, row_spec, row_spec],
        out_specs=pl.BlockSpec((B, TN), lambda n: (0, n)),
        compiler_params=pltpu.CompilerParams(
            dimension_semantics=("parallel",),
            vmem_limit_bytes=60 * 1024 * 1024,
        ),
    )(x, cgamma_t, weight_mu_t, weight_sigma_t, eps_w_t,
      bias_mu_row, bias_sigma_row, eps_b)
